# Initial kernel scaffold; baseline (speedup 1.0000x reference)
import jax
import jax.numpy as jnp
from jax.experimental import pallas as pl


def kernel(dist, angle, idx_kj):
    raise NotImplementedError("write your pallas kernel here")


# speedup vs baseline: 1.4398x; 1.4398x over previous
import functools

import jax
import jax.numpy as jnp
import numpy as np
from jax import lax
from jax.experimental import pallas as pl
from jax.experimental.pallas import tpu as pltpu
import jax.experimental.pallas.tpu_sc as plsc

NUM_SPH = 7
NUM_RAD = 6
CUTOFF = 5.0
P_ENV = 6
A_ENV = -(P_ENV + 1) * (P_ENV + 2) / 2.0
B_ENV = float(P_ENV * (P_ENV + 2))
C_ENV = -P_ENV * (P_ENV + 1) / 2.0

NCOL = NUM_SPH * NUM_RAD
NPAD = 48


def _jn_np(x, l):
    x = np.asarray(x, dtype=np.float64)
    j0 = np.sin(x) / x
    if l == 0:
        return j0
    j1 = np.sin(x) / x ** 2 - np.cos(x) / x
    if l == 1:
        return j1
    jm1, j = j0, j1
    for ll in range(1, l):
        jm1, j = j, (2 * ll + 1) / x * j - jm1
    return j


def _jn_zeros(n, k):
    zeros = np.zeros((n, k))
    for l in range(n):
        grid = np.linspace(0.5 + 0.5 * l, 45.0 + 5.0 * l, 200001)
        vals = _jn_np(grid, l)
        sign_change = np.where(np.sign(vals[:-1]) * np.sign(vals[1:]) < 0)[0][:k]
        for i, ii in enumerate(sign_change):
            lo, hi = grid[ii], grid[ii + 1]
            flo = _jn_np(lo, l)
            for _ in range(60):
                mid = 0.5 * (lo + hi)
                fmid = _jn_np(mid, l)
                if flo * fmid <= 0:
                    hi = mid
                else:
                    lo, flo = mid, fmid
            zeros[l, i] = 0.5 * (lo + hi)
    return zeros


_Z = _jn_zeros(NUM_SPH, NUM_RAD)
_NORM = np.stack([1.0 / np.sqrt(0.5 * _jn_np(_Z[l], l + 1) ** 2) for l in range(NUM_SPH)])
_SPH_PREF = np.array([np.sqrt((2 * l + 1) / (4 * np.pi)) for l in range(NUM_SPH)])

_ZPAD = np.ones((NPAD,), np.float32)
_ZPAD[:NCOL] = _Z.reshape(-1)
_NORMPAD = np.zeros((NPAD,), np.float32)
_NORMPAD[:NCOL] = _NORM.reshape(-1)

_LMAP = [[(16 * v + k) // NUM_RAD for k in range(16)] for v in range(3)]


def _rbf_body(z_ref, n_ref, d_ref, o_ref):
    x = d_ref[:] * (1.0 / CUTOFF)
    x2 = x * x
    x4 = x2 * x2
    x5 = x4 * x
    env = 1.0 / x + A_ENV * x5 + B_ENV * x5 * x + C_ENV * x5 * x2
    y = x[:, None] * z_ref[:]
    inv_y = 1.0 / y
    sy = jnp.sin(y)
    cy = jnp.cos(y)
    j0 = sy * inv_y
    j1 = sy * inv_y * inv_y - cy * inv_y
    lcol = lax.broadcasted_iota(jnp.int32, y.shape, 1) // NUM_RAD
    res = jnp.where(lcol == 0, j0, j1)
    jm1, j = j0, j1
    for s in range(1, NUM_SPH - 1):
        jm1, j = j, (2 * s + 1) * inv_y * j - jm1
        res = jnp.where(lcol == s + 1, j, res)
    rbf = res * n_ref[:] * env[:, None]
    o_ref[:, :] = jnp.where(lcol >= NUM_SPH, 0.0, rbf)


def _cbf_body(a_ref, o_ref):
    z = jnp.cos(a_ref[:])
    ps = [jnp.ones_like(z), z]
    for l in range(1, NUM_SPH - 1):
        ps.append(((2 * l + 1) * z * ps[-1] - l * ps[-2]) / (l + 1))
    lcol = lax.broadcasted_iota(jnp.int32, (a_ref.shape[0], 8), 1)
    res = jnp.zeros((a_ref.shape[0], 8), jnp.float32)
    for l in range(NUM_SPH):
        res = jnp.where(lcol == l, float(_SPH_PREF[l]) * ps[l][:, None], res)
    o_ref[:, :] = res


def _make_sc_gather(E, T):
    NC = 2
    NS = 16
    NW = NC * NS
    ROWS_W = T // NW
    SUB = 120
    CH = 600
    NSUB = CH // SUB
    NCHUNK = ROWS_W // CH
    assert T == NW * NCHUNK * CH and CH == NSUB * SUB

    mesh = plsc.VectorSubcoreMesh(core_axis_name="c", subcore_axis_name="s")

    @functools.partial(
        pl.kernel,
        out_type=jax.ShapeDtypeStruct((T * NCOL,), jnp.float32),
        mesh=mesh,
        scratch_types=[
            pltpu.VMEM((CH,), jnp.int32),
            pltpu.VMEM((CH, NPAD), jnp.float32),
            pltpu.VMEM((CH * 8 + 16,), jnp.float32),
            pltpu.VMEM((CH * NCOL + 16,), jnp.float32),
            pltpu.SemaphoreType.DMA,
        ],
        compiler_params=pltpu.CompilerParams(
            use_tc_tiling_on_sc=False, needs_layout_passes=False
        ),
    )
    def sc_gather_mul(rbf_hbm, cbf_hbm, idx_hbm, out_hbm, idx_v, rows_v, cbf_v, out_v, sem):
        wid = lax.axis_index("s") * NC + lax.axis_index("c")
        row0 = wid * ROWS_W
        lane = lax.iota(jnp.int32, 16)
        lm0 = lax.shift_right_logical(lane * 43, 8)
        lm1 = lax.shift_right_logical((lane + 16) * 43, 8)
        lm2 = lax.shift_right_logical((lane + 32) * 43, 8)

        def chunk_body(i, carry):
            base = row0 + i * CH
            pltpu.sync_copy(idx_hbm.at[pl.ds(base, CH)], idx_v)
            cps = [
                pltpu.async_copy(
                    rbf_hbm.at[idx_v.at[pl.ds(j * SUB, SUB)]],
                    rows_v.at[pl.ds(j * SUB, SUB)],
                    sem,
                )
                for j in range(NSUB)
            ]
            pltpu.sync_copy(cbf_hbm.at[pl.ds(base * 8, CH * 8)], cbf_v.at[pl.ds(0, CH * 8)])
            for cp in cps:
                cp.wait()

            def row_body(r, c2):
                rb = r * 8
                w0 = plsc.load_gather(cbf_v, [rb + lm0])
                w1 = plsc.load_gather(cbf_v, [rb + lm1])
                w2 = plsc.load_gather(cbf_v, [rb + lm2])
                m0 = rows_v[r, pl.ds(0, 16)]
                m1 = rows_v[r, pl.ds(16, 16)]
                m2 = rows_v[r, pl.ds(32, 16)]
                ob = r * NCOL
                out_v[pl.ds(ob, 16)] = m0 * w0
                out_v[pl.ds(ob + 16, 16)] = m1 * w1
                out_v[pl.ds(ob + 32, 16)] = m2 * w2
                return c2

            lax.fori_loop(0, CH, row_body, 0)
            pltpu.sync_copy(
                out_v.at[pl.ds(0, CH * NCOL)],
                out_hbm.at[pl.ds(base * NCOL, CH * NCOL)],
            )
            return carry

        lax.fori_loop(0, NCHUNK, chunk_body, 0)

    return sc_gather_mul


def kernel(dist, angle, idx_kj):
    E = dist.shape[0]
    T = angle.shape[0]

    BE = 512
    rbf = pl.pallas_call(
        _rbf_body,
        grid=(E // BE,),
        in_specs=[
            pl.BlockSpec((1, NPAD), lambda i: (0, 0)),
            pl.BlockSpec((1, NPAD), lambda i: (0, 0)),
            pl.BlockSpec((BE,), lambda i: (i,)),
        ],
        out_specs=pl.BlockSpec((BE, NPAD), lambda i: (i, 0)),
        out_shape=jax.ShapeDtypeStruct((E, NPAD), jnp.float32),
    )(jnp.asarray(_ZPAD)[None, :], jnp.asarray(_NORMPAD)[None, :], dist)

    BT = 512
    cbf = pl.pallas_call(
        _cbf_body,
        grid=(T // BT,),
        in_specs=[pl.BlockSpec((BT,), lambda i: (i,))],
        out_specs=pl.BlockSpec((BT, 8), lambda i: (i, 0)),
        out_shape=jax.ShapeDtypeStruct((T, 8), jnp.float32),
    )(angle)

    sc_gather_mul = _make_sc_gather(E, T)
    out_flat = sc_gather_mul(rbf, cbf.reshape(-1), idx_kj)
    return out_flat.reshape(T, NCOL)


# speedup vs baseline: 1.9336x; 1.3430x over previous
import functools

import jax
import jax.numpy as jnp
import numpy as np
from jax import lax
from jax.experimental import pallas as pl
from jax.experimental.pallas import tpu as pltpu
import jax.experimental.pallas.tpu_sc as plsc

NUM_SPH = 7
NUM_RAD = 6
CUTOFF = 5.0
P_ENV = 6
A_ENV = -(P_ENV + 1) * (P_ENV + 2) / 2.0
B_ENV = float(P_ENV * (P_ENV + 2))
C_ENV = -P_ENV * (P_ENV + 1) / 2.0

NCOL = NUM_SPH * NUM_RAD
NPAD = 48


def _jn_np(x, l):
    x = np.asarray(x, dtype=np.float64)
    j0 = np.sin(x) / x
    if l == 0:
        return j0
    j1 = np.sin(x) / x ** 2 - np.cos(x) / x
    if l == 1:
        return j1
    jm1, j = j0, j1
    for ll in range(1, l):
        jm1, j = j, (2 * ll + 1) / x * j - jm1
    return j


def _jn_zeros(n, k):
    zeros = np.zeros((n, k))
    for l in range(n):
        grid = np.linspace(0.5 + 0.5 * l, 45.0 + 5.0 * l, 200001)
        vals = _jn_np(grid, l)
        sign_change = np.where(np.sign(vals[:-1]) * np.sign(vals[1:]) < 0)[0][:k]
        for i, ii in enumerate(sign_change):
            lo, hi = grid[ii], grid[ii + 1]
            flo = _jn_np(lo, l)
            for _ in range(60):
                mid = 0.5 * (lo + hi)
                fmid = _jn_np(mid, l)
                if flo * fmid <= 0:
                    hi = mid
                else:
                    lo, flo = mid, fmid
            zeros[l, i] = 0.5 * (lo + hi)
    return zeros


_Z = _jn_zeros(NUM_SPH, NUM_RAD)
_NORM = np.stack([1.0 / np.sqrt(0.5 * _jn_np(_Z[l], l + 1) ** 2) for l in range(NUM_SPH)])
_SPH_PREF = np.array([np.sqrt((2 * l + 1) / (4 * np.pi)) for l in range(NUM_SPH)])

_ZPAD = np.ones((NPAD,), np.float32)
_ZPAD[:NCOL] = _Z.reshape(-1)
_NORMPAD = np.zeros((NPAD,), np.float32)
_NORMPAD[:NCOL] = _NORM.reshape(-1)
for _c in range(NCOL):
    _NORMPAD[_c] *= np.float32(_SPH_PREF[_c // NUM_RAD])


def _make_sc_gather(E, T):
    NC = 2
    NW = 32
    ROWS_W = T // NW
    SUB = 120
    CH = 3000
    NSUB = CH // SUB
    NCHUNK = ROWS_W // CH
    assert T == NW * NCHUNK * CH and CH == NSUB * SUB

    mesh = plsc.VectorSubcoreMesh(core_axis_name="c", subcore_axis_name="s")

    @functools.partial(
        pl.kernel,
        out_type=jax.ShapeDtypeStruct((T,), jnp.float32),
        mesh=mesh,
        scratch_types=[
            pltpu.VMEM((CH,), jnp.int32),
            pltpu.VMEM((CH,), jnp.float32),
            pltpu.SemaphoreType.DMA,
        ],
    )
    def sc_gather(dist_hbm, idx_hbm, out_hbm, idx_v, val_v, sem):
        wid = lax.axis_index("s") * NC + lax.axis_index("c")
        row0 = wid * ROWS_W

        def chunk_body(i, carry):
            base = row0 + i * CH
            pltpu.sync_copy(idx_hbm.at[pl.ds(base, CH)], idx_v)
            cps = [
                pltpu.async_copy(
                    dist_hbm.at[idx_v.at[pl.ds(j * SUB, SUB)]],
                    val_v.at[pl.ds(j * SUB, SUB)],
                    sem,
                )
                for j in range(NSUB)
            ]
            for cp in cps:
                cp.wait()
            pltpu.sync_copy(val_v, out_hbm.at[pl.ds(base, CH)])
            return carry

        lax.fori_loop(0, NCHUNK, chunk_body, 0)

    return sc_gather


def _basis_body(z_ref, n_ref, d_ref, a_ref, o_ref):
    x = d_ref[:] * (1.0 / CUTOFF)
    x2 = x * x
    x4 = x2 * x2
    x5 = x4 * x
    env = 1.0 / x + A_ENV * x5 + B_ENV * x5 * x + C_ENV * x5 * x2
    y = x[:, None] * z_ref[:]
    inv_y = 1.0 / y
    sy = jnp.sin(y)
    cy = jnp.cos(y)
    j0 = sy * inv_y
    j1 = sy * inv_y * inv_y - cy * inv_y
    lcol = lax.broadcasted_iota(jnp.int32, y.shape, 1) // NUM_RAD
    res = jnp.where(lcol == 0, j0, j1)
    jm1, j = j0, j1
    for s in range(1, NUM_SPH - 1):
        jm1, j = j, (2 * s + 1) * inv_y * j - jm1
        res = jnp.where(lcol == s + 1, j, res)

    zc = jnp.cos(a_ref[:])
    ps = [jnp.ones_like(zc), zc]
    for l in range(1, NUM_SPH - 1):
        ps.append(((2 * l + 1) * zc * ps[-1] - l * ps[-2]) / (l + 1))
    cb = ps[0][:, None]
    for l in range(1, NUM_SPH):
        cb = jnp.where(lcol == l, ps[l][:, None], cb)

    o_ref[:, :] = (res * n_ref[:] * env[:, None] * cb)[:, :NCOL]


def kernel(dist, angle, idx_kj):
    E = dist.shape[0]
    T = angle.shape[0]

    sc_gather = _make_sc_gather(E, T)
    dist_t = sc_gather(dist, idx_kj)

    BT = 1024
    out = pl.pallas_call(
        _basis_body,
        grid=(pl.cdiv(T, BT),),
        in_specs=[
            pl.BlockSpec((1, NPAD), lambda i: (0, 0)),
            pl.BlockSpec((1, NPAD), lambda i: (0, 0)),
            pl.BlockSpec((BT,), lambda i: (i,)),
            pl.BlockSpec((BT,), lambda i: (i,)),
        ],
        out_specs=pl.BlockSpec((BT, NCOL), lambda i: (i, 0)),
        out_shape=jax.ShapeDtypeStruct((T, NCOL), jnp.float32),
    )(jnp.asarray(_ZPAD)[None, :], jnp.asarray(_NORMPAD)[None, :], dist_t, angle)
    return out


# speedup vs baseline: 5.6247x; 2.9089x over previous
import functools

import jax
import jax.numpy as jnp
import numpy as np
from jax import lax
from jax.experimental import pallas as pl
from jax.experimental.pallas import tpu as pltpu
import jax.experimental.pallas.tpu_sc as plsc

NUM_SPH = 7
NUM_RAD = 6
CUTOFF = 5.0
P_ENV = 6
A_ENV = -(P_ENV + 1) * (P_ENV + 2) / 2.0
B_ENV = float(P_ENV * (P_ENV + 2))
C_ENV = -P_ENV * (P_ENV + 1) / 2.0

NCOL = NUM_SPH * NUM_RAD
NPAD = 48


def _jn_np(x, l):
    x = np.asarray(x, dtype=np.float64)
    j0 = np.sin(x) / x
    if l == 0:
        return j0
    j1 = np.sin(x) / x ** 2 - np.cos(x) / x
    if l == 1:
        return j1
    jm1, j = j0, j1
    for ll in range(1, l):
        jm1, j = j, (2 * ll + 1) / x * j - jm1
    return j


def _jn_zeros(n, k):
    zeros = np.zeros((n, k))
    for l in range(n):
        grid = np.linspace(0.5 + 0.5 * l, 45.0 + 5.0 * l, 200001)
        vals = _jn_np(grid, l)
        sign_change = np.where(np.sign(vals[:-1]) * np.sign(vals[1:]) < 0)[0][:k]
        for i, ii in enumerate(sign_change):
            lo, hi = grid[ii], grid[ii + 1]
            flo = _jn_np(lo, l)
            for _ in range(60):
                mid = 0.5 * (lo + hi)
                fmid = _jn_np(mid, l)
                if flo * fmid <= 0:
                    hi = mid
                else:
                    lo, flo = mid, fmid
            zeros[l, i] = 0.5 * (lo + hi)
    return zeros


_Z = _jn_zeros(NUM_SPH, NUM_RAD)
_NORM = np.stack([1.0 / np.sqrt(0.5 * _jn_np(_Z[l], l + 1) ** 2) for l in range(NUM_SPH)])
_SPH_PREF = np.array([np.sqrt((2 * l + 1) / (4 * np.pi)) for l in range(NUM_SPH)])

_ZPAD = np.ones((NPAD,), np.float32)
_ZPAD[:NCOL] = _Z.reshape(-1)
_NORMPAD = np.zeros((NPAD,), np.float32)
_NORMPAD[:NCOL] = _NORM.reshape(-1)
for _c in range(NCOL):
    _NORMPAD[_c] *= np.float32(_SPH_PREF[_c // NUM_RAD])


def _make_sc_gather(E, T):
    NC = 2
    NW = 32
    ROWS_W = T // NW
    SUB = 120
    CH = 3000
    NSUB = CH // SUB
    NCHUNK = ROWS_W // CH
    assert T == NW * NCHUNK * CH and CH == NSUB * SUB

    mesh = plsc.VectorSubcoreMesh(core_axis_name="c", subcore_axis_name="s")

    @functools.partial(
        pl.kernel,
        out_type=jax.ShapeDtypeStruct((T,), jnp.float32),
        mesh=mesh,
        scratch_types=[
            pltpu.VMEM((CH,), jnp.int32),
            pltpu.VMEM((CH,), jnp.float32),
            pltpu.SemaphoreType.DMA,
        ],
    )
    def sc_gather(dist_hbm, idx_hbm, out_hbm, idx_v, val_v, sem):
        wid = lax.axis_index("s") * NC + lax.axis_index("c")
        row0 = wid * ROWS_W

        def chunk_body(i, carry):
            base = row0 + i * CH
            pltpu.sync_copy(idx_hbm.at[pl.ds(base, CH)], idx_v)
            cps = [
                pltpu.async_copy(
                    dist_hbm.at[idx_v.at[pl.ds(j * SUB, SUB)]],
                    val_v.at[pl.ds(j * SUB, SUB)],
                    sem,
                )
                for j in range(NSUB)
            ]
            for cp in cps:
                cp.wait()
            pltpu.sync_copy(val_v, out_hbm.at[pl.ds(base, CH)])
            return carry

        lax.fori_loop(0, NCHUNK, chunk_body, 0)

    return sc_gather


_PIO2_HI = np.float32(1.57079625129699707031)
_PIO2_LO = np.float32(7.54978941586159635335e-08)
_S1, _S2, _S3 = np.float32(-1.6666654611e-1), np.float32(8.3321608736e-3), np.float32(-1.9515295891e-4)
_C1, _C2, _C3 = np.float32(-0.5), np.float32(4.166664568298827e-2), np.float32(-1.388731625493765e-3)
_C4 = np.float32(2.443315711809948e-5)


def _sincos(y):
    q = y * np.float32(2.0 / np.pi)
    nf = jnp.floor(q + 0.5)
    ni = nf.astype(jnp.int32)
    r = (y - nf * _PIO2_HI) - nf * _PIO2_LO
    r2 = r * r
    sp = r + r * r2 * (_S1 + r2 * (_S2 + r2 * _S3))
    cp = 1.0 + r2 * (_C1 + r2 * (_C2 + r2 * (_C3 + r2 * _C4)))
    odd = (ni & 1) == 1
    sin_sel = jnp.where(odd, cp, sp)
    cos_sel = jnp.where(odd, sp, cp)
    sbit = jax.lax.shift_left(ni & 2, 30)
    cbit = jax.lax.shift_left((ni + 1) & 2, 30)
    sin = lax.bitcast_convert_type(
        lax.bitcast_convert_type(sin_sel, jnp.int32) ^ sbit, jnp.float32
    )
    cos = lax.bitcast_convert_type(
        lax.bitcast_convert_type(cos_sel, jnp.int32) ^ cbit, jnp.float32
    )
    return sin, cos


def _basis_body(z_ref, n_ref, d_ref, a_ref, o_ref):
    x = d_ref[:] * (1.0 / CUTOFF)
    x2 = x * x
    x4 = x2 * x2
    x5 = x4 * x
    env = 1.0 / x + A_ENV * x5 + B_ENV * x5 * x + C_ENV * x5 * x2
    y = z_ref[:] * x[None, :]
    inv_y = 1.0 / y
    sy, cy = _sincos(y)
    j0 = sy * inv_y
    j1 = sy * inv_y * inv_y - cy * inv_y
    lcol = lax.broadcasted_iota(jnp.int32, y.shape, 0) // NUM_RAD
    res = jnp.where(lcol == 0, j0, j1)
    jm1, j = j0, j1
    for s in range(1, NUM_SPH - 1):
        jm1, j = j, (2 * s + 1) * inv_y * j - jm1
        res = jnp.where(lcol == s + 1, j, res)

    zc = _sincos(a_ref[:])[1]
    ps = [jnp.ones_like(zc), zc]
    for l in range(1, NUM_SPH - 1):
        ps.append(((2 * l + 1) * zc * ps[-1] - l * ps[-2]) / (l + 1))
    cb = ps[0][None, :]
    for l in range(1, NUM_SPH):
        cb = jnp.where(lcol == l, ps[l][None, :], cb)

    full = res * n_ref[:] * env[None, :] * cb
    o_ref[:, :] = full.T[:, :NCOL]


def kernel(dist, angle, idx_kj):
    E = dist.shape[0]
    T = angle.shape[0]

    sc_gather = _make_sc_gather(E, T)
    dist_t = sc_gather(dist, idx_kj)

    BT = 2048
    out = pl.pallas_call(
        _basis_body,
        grid=(pl.cdiv(T, BT),),
        in_specs=[
            pl.BlockSpec((NPAD, 1), lambda i: (0, 0)),
            pl.BlockSpec((NPAD, 1), lambda i: (0, 0)),
            pl.BlockSpec((BT,), lambda i: (i,)),
            pl.BlockSpec((BT,), lambda i: (i,)),
        ],
        out_specs=pl.BlockSpec((BT, NCOL), lambda i: (i, 0)),
        out_shape=jax.ShapeDtypeStruct((T, NCOL), jnp.float32),
    )(jnp.asarray(_ZPAD)[:, None], jnp.asarray(_NORMPAD)[:, None], dist_t, angle)
    return out


# speedup vs baseline: 8.2444x; 1.4657x over previous
import functools

import jax
import jax.numpy as jnp
import numpy as np
from jax import lax
from jax.experimental import pallas as pl
from jax.experimental.pallas import tpu as pltpu
import jax.experimental.pallas.tpu_sc as plsc

NUM_SPH = 7
NUM_RAD = 6
CUTOFF = 5.0
P_ENV = 6
A_ENV = -(P_ENV + 1) * (P_ENV + 2) / 2.0
B_ENV = float(P_ENV * (P_ENV + 2))
C_ENV = -P_ENV * (P_ENV + 1) / 2.0

NCOL = NUM_SPH * NUM_RAD
NPAD = 48


def _jn_np(x, l):
    x = np.asarray(x, dtype=np.float64)
    j0 = np.sin(x) / x
    if l == 0:
        return j0
    j1 = np.sin(x) / x ** 2 - np.cos(x) / x
    if l == 1:
        return j1
    jm1, j = j0, j1
    for ll in range(1, l):
        jm1, j = j, (2 * ll + 1) / x * j - jm1
    return j


def _jn_zeros(n, k):
    zeros = np.zeros((n, k))
    for l in range(n):
        grid = np.linspace(0.5 + 0.5 * l, 45.0 + 5.0 * l, 200001)
        vals = _jn_np(grid, l)
        sign_change = np.where(np.sign(vals[:-1]) * np.sign(vals[1:]) < 0)[0][:k]
        for i, ii in enumerate(sign_change):
            lo, hi = grid[ii], grid[ii + 1]
            flo = _jn_np(lo, l)
            for _ in range(60):
                mid = 0.5 * (lo + hi)
                fmid = _jn_np(mid, l)
                if flo * fmid <= 0:
                    hi = mid
                else:
                    lo, flo = mid, fmid
            zeros[l, i] = 0.5 * (lo + hi)
    return zeros


_Z = _jn_zeros(NUM_SPH, NUM_RAD)
_NORM = np.stack([1.0 / np.sqrt(0.5 * _jn_np(_Z[l], l + 1) ** 2) for l in range(NUM_SPH)])
_SPH_PREF = np.array([np.sqrt((2 * l + 1) / (4 * np.pi)) for l in range(NUM_SPH)])

_ZPAD = np.ones((NPAD,), np.float32)
_ZPAD[:NCOL] = _Z.reshape(-1)
_NORMPAD = np.zeros((NPAD,), np.float32)
_NORMPAD[:NCOL] = _NORM.reshape(-1)
for _c in range(NCOL):
    _NORMPAD[_c] *= np.float32(_SPH_PREF[_c // NUM_RAD])


def _make_sc_gather(E, T):
    NC = 2
    NW = 32
    ROWS_W = T // NW
    SUB = 120
    CH = 3000
    NSUB = CH // SUB
    NCHUNK = ROWS_W // CH
    assert T == NW * NCHUNK * CH and CH == NSUB * SUB

    mesh = plsc.VectorSubcoreMesh(core_axis_name="c", subcore_axis_name="s")

    @functools.partial(
        pl.kernel,
        out_type=jax.ShapeDtypeStruct((T,), jnp.float32),
        mesh=mesh,
        scratch_types=[
            pltpu.VMEM((CH,), jnp.int32),
            pltpu.VMEM((CH,), jnp.float32),
            pltpu.SemaphoreType.DMA,
        ],
    )
    def sc_gather(dist_hbm, idx_hbm, out_hbm, idx_v, val_v, sem):
        wid = lax.axis_index("s") * NC + lax.axis_index("c")
        row0 = wid * ROWS_W

        def chunk_body(i, carry):
            base = row0 + i * CH
            pltpu.sync_copy(idx_hbm.at[pl.ds(base, CH)], idx_v)
            cps = [
                pltpu.async_copy(
                    dist_hbm.at[idx_v.at[pl.ds(j * SUB, SUB)]],
                    val_v.at[pl.ds(j * SUB, SUB)],
                    sem,
                )
                for j in range(NSUB)
            ]
            for cp in cps:
                cp.wait()
            pltpu.sync_copy(val_v, out_hbm.at[pl.ds(base, CH)])
            return carry

        lax.fori_loop(0, NCHUNK, chunk_body, 0)

    return sc_gather


_PIO2_HI = np.float32(1.57079625129699707031)
_PIO2_LO = np.float32(7.54978941586159635335e-08)
_S1, _S2, _S3 = np.float32(-1.6666654611e-1), np.float32(8.3321608736e-3), np.float32(-1.9515295891e-4)
_C1, _C2, _C3 = np.float32(-0.5), np.float32(4.166664568298827e-2), np.float32(-1.388731625493765e-3)
_C4 = np.float32(2.443315711809948e-5)


def _sincos(y):
    q = y * np.float32(2.0 / np.pi)
    nf = jnp.floor(q + 0.5)
    ni = nf.astype(jnp.int32)
    r = (y - nf * _PIO2_HI) - nf * _PIO2_LO
    r2 = r * r
    sp = r + r * r2 * (_S1 + r2 * (_S2 + r2 * _S3))
    cp = 1.0 + r2 * (_C1 + r2 * (_C2 + r2 * (_C3 + r2 * _C4)))
    odd = (ni & 1) == 1
    sin_sel = jnp.where(odd, cp, sp)
    cos_sel = jnp.where(odd, sp, cp)
    sbit = jax.lax.shift_left(ni & 2, 30)
    cbit = jax.lax.shift_left((ni + 1) & 2, 30)
    sin = lax.bitcast_convert_type(
        lax.bitcast_convert_type(sin_sel, jnp.int32) ^ sbit, jnp.float32
    )
    cos = lax.bitcast_convert_type(
        lax.bitcast_convert_type(cos_sel, jnp.int32) ^ cbit, jnp.float32
    )
    return sin, cos


def _basis_body(z_ref, n_ref, d_ref, a_ref, o_ref):
    for k in range(d_ref.shape[0] // 128):
        sl = pl.ds(k * 128, 128)
        _basis_chunk(z_ref, n_ref, d_ref[sl], a_ref[sl], o_ref, sl)


def _basis_chunk(z_ref, n_ref, d, a, o_ref, sl):
    x = d * (1.0 / CUTOFF)
    x2 = x * x
    x4 = x2 * x2
    x5 = x4 * x
    env = 1.0 / x + A_ENV * x5 + B_ENV * x5 * x + C_ENV * x5 * x2
    y = z_ref[:] * x[None, :]
    inv_y = 1.0 / y
    sy, cy = _sincos(y)
    j0 = sy * inv_y
    j1 = sy * inv_y * inv_y - cy * inv_y
    lcol = lax.broadcasted_iota(jnp.int32, y.shape, 0) // NUM_RAD
    res = jnp.where(lcol == 0, j0, j1)
    jm1, j = j0, j1
    for s in range(1, NUM_SPH - 1):
        jm1, j = j, (2 * s + 1) * inv_y * j - jm1
        res = jnp.where(lcol == s + 1, j, res)

    zc = _sincos(a)[1]
    ps = [jnp.ones_like(zc), zc]
    for l in range(1, NUM_SPH - 1):
        ps.append(((2 * l + 1) * zc * ps[-1] - l * ps[-2]) / (l + 1))
    cb = ps[0][None, :]
    for l in range(1, NUM_SPH):
        cb = jnp.where(lcol == l, ps[l][None, :], cb)

    full = res * n_ref[:] * env[None, :] * cb
    o_ref[:, sl] = full[:NCOL, :]


def kernel(dist, angle, idx_kj):
    E = dist.shape[0]
    T = angle.shape[0]

    sc_gather = _make_sc_gather(E, T)
    dist_t = sc_gather(dist, idx_kj)

    BT = 2048
    out = pl.pallas_call(
        _basis_body,
        grid=(pl.cdiv(T, BT),),
        in_specs=[
            pl.BlockSpec((NPAD, 1), lambda i: (0, 0)),
            pl.BlockSpec((NPAD, 1), lambda i: (0, 0)),
            pl.BlockSpec((BT,), lambda i: (i,)),
            pl.BlockSpec((BT,), lambda i: (i,)),
        ],
        out_specs=pl.BlockSpec((NCOL, BT), lambda i: (0, i)),
        out_shape=jax.ShapeDtypeStruct((NCOL, T), jnp.float32),
    )(jnp.asarray(_ZPAD)[:, None], jnp.asarray(_NORMPAD)[:, None], dist_t, angle)
    return out.T


# speedup vs baseline: 9.2588x; 1.1230x over previous
import functools

import jax
import jax.numpy as jnp
import numpy as np
from jax import lax
from jax.experimental import pallas as pl
from jax.experimental.pallas import tpu as pltpu
import jax.experimental.pallas.tpu_sc as plsc

NUM_SPH = 7
NUM_RAD = 6
CUTOFF = 5.0
P_ENV = 6
A_ENV = -(P_ENV + 1) * (P_ENV + 2) / 2.0
B_ENV = float(P_ENV * (P_ENV + 2))
C_ENV = -P_ENV * (P_ENV + 1) / 2.0

NCOL = NUM_SPH * NUM_RAD
NPAD = 48


def _jn_np(x, l):
    x = np.asarray(x, dtype=np.float64)
    j0 = np.sin(x) / x
    if l == 0:
        return j0
    j1 = np.sin(x) / x ** 2 - np.cos(x) / x
    if l == 1:
        return j1
    jm1, j = j0, j1
    for ll in range(1, l):
        jm1, j = j, (2 * ll + 1) / x * j - jm1
    return j


def _jn_zeros(n, k):
    zeros = np.zeros((n, k))
    for l in range(n):
        grid = np.linspace(0.5 + 0.5 * l, 45.0 + 5.0 * l, 200001)
        vals = _jn_np(grid, l)
        sign_change = np.where(np.sign(vals[:-1]) * np.sign(vals[1:]) < 0)[0][:k]
        for i, ii in enumerate(sign_change):
            lo, hi = grid[ii], grid[ii + 1]
            flo = _jn_np(lo, l)
            for _ in range(60):
                mid = 0.5 * (lo + hi)
                fmid = _jn_np(mid, l)
                if flo * fmid <= 0:
                    hi = mid
                else:
                    lo, flo = mid, fmid
            zeros[l, i] = 0.5 * (lo + hi)
    return zeros


_Z = _jn_zeros(NUM_SPH, NUM_RAD)
_NORM = np.stack([1.0 / np.sqrt(0.5 * _jn_np(_Z[l], l + 1) ** 2) for l in range(NUM_SPH)])
_SPH_PREF = np.array([np.sqrt((2 * l + 1) / (4 * np.pi)) for l in range(NUM_SPH)])

_ZPAD = np.ones((NPAD,), np.float32)
_ZPAD[:NCOL] = _Z.reshape(-1)
_NORMPAD = np.zeros((NPAD,), np.float32)
_NORMPAD[:NCOL] = _NORM.reshape(-1)
for _c in range(NCOL):
    _NORMPAD[_c] *= np.float32(_SPH_PREF[_c // NUM_RAD])

_PLC = np.zeros((NUM_SPH, NUM_SPH))
_PLC[0, 0] = 1.0
_PLC[1, 1] = 1.0
for _l in range(1, NUM_SPH - 1):
    _PLC[_l + 1, 1:] = (2 * _l + 1) * _PLC[_l, :6]
    _PLC[_l + 1] -= _l * _PLC[_l - 1]
    _PLC[_l + 1] /= _l + 1
_PLPAD = np.zeros((NPAD, 8), np.float32)
for _c in range(NCOL):
    _PLPAD[_c, :NUM_SPH] = _PLC[_c // NUM_RAD].astype(np.float32)


def _make_sc_gather(E, T):
    NC = 2
    NW = 32
    ROWS_W = T // NW
    SUB = 120
    CH = 3000
    NSUB = CH // SUB
    NCHUNK = ROWS_W // CH
    assert T == NW * NCHUNK * CH and CH == NSUB * SUB

    mesh = plsc.VectorSubcoreMesh(core_axis_name="c", subcore_axis_name="s")

    @functools.partial(
        pl.kernel,
        out_type=jax.ShapeDtypeStruct((T,), jnp.float32),
        mesh=mesh,
        scratch_types=[
            pltpu.VMEM((CH,), jnp.int32),
            pltpu.VMEM((CH,), jnp.float32),
            pltpu.SemaphoreType.DMA,
        ],
    )
    def sc_gather(dist_hbm, idx_hbm, out_hbm, idx_v, val_v, sem):
        wid = lax.axis_index("s") * NC + lax.axis_index("c")
        row0 = wid * ROWS_W

        def chunk_body(i, carry):
            base = row0 + i * CH
            pltpu.sync_copy(idx_hbm.at[pl.ds(base, CH)], idx_v)
            cps = [
                pltpu.async_copy(
                    dist_hbm.at[idx_v.at[pl.ds(j * SUB, SUB)]],
                    val_v.at[pl.ds(j * SUB, SUB)],
                    sem,
                )
                for j in range(NSUB)
            ]
            for cp in cps:
                cp.wait()
            pltpu.sync_copy(val_v, out_hbm.at[pl.ds(base, CH)])
            return carry

        lax.fori_loop(0, NCHUNK, chunk_body, 0)

    return sc_gather


_PIO2_HI = np.float32(1.57079625129699707031)
_PIO2_LO = np.float32(7.54978941586159635335e-08)
_S1, _S2, _S3 = np.float32(-1.6666654611e-1), np.float32(8.3321608736e-3), np.float32(-1.9515295891e-4)
_C1, _C2, _C3 = np.float32(-0.5), np.float32(4.166664568298827e-2), np.float32(-1.388731625493765e-3)
_C4 = np.float32(2.443315711809948e-5)


def _sincos(y):
    q = y * np.float32(2.0 / np.pi)
    nf = jnp.floor(q + 0.5)
    ni = nf.astype(jnp.int32)
    r = (y - nf * _PIO2_HI) - nf * _PIO2_LO
    r2 = r * r
    sp = r + r * r2 * (_S1 + r2 * (_S2 + r2 * _S3))
    cp = 1.0 + r2 * (_C1 + r2 * (_C2 + r2 * (_C3 + r2 * _C4)))
    odd = (ni & 1) == 1
    sin_sel = jnp.where(odd, cp, sp)
    cos_sel = jnp.where(odd, sp, cp)
    sbit = jax.lax.shift_left(ni & 2, 30)
    cbit = jax.lax.shift_left((ni + 1) & 2, 30)
    sin = lax.bitcast_convert_type(
        lax.bitcast_convert_type(sin_sel, jnp.int32) ^ sbit, jnp.float32
    )
    cos = lax.bitcast_convert_type(
        lax.bitcast_convert_type(cos_sel, jnp.int32) ^ cbit, jnp.float32
    )
    return sin, cos


def _basis_body(z_ref, n_ref, p_ref, d_ref, a_ref, o_ref):
    for k in range(d_ref.shape[0] // 128):
        sl = pl.ds(k * 128, 128)
        _basis_chunk(z_ref, n_ref, p_ref, d_ref[sl], a_ref[sl], o_ref, sl)


def _basis_chunk(z_ref, n_ref, p_ref, d, a, o_ref, sl):
    x = d * (1.0 / CUTOFF)
    x2 = x * x
    x4 = x2 * x2
    x5 = x4 * x
    env = 1.0 / x + A_ENV * x5 + B_ENV * x5 * x + C_ENV * x5 * x2
    y = z_ref[:] * x[None, :]
    inv_y = 1.0 / y
    sy, cy = _sincos(y)
    j0 = sy * inv_y
    j1 = sy * inv_y * inv_y - cy * inv_y
    lcol = lax.broadcasted_iota(jnp.int32, y.shape, 0) // NUM_RAD
    res = jnp.where(lcol == 0, j0, j1)
    jm1, j = j0, j1
    for s in range(1, NUM_SPH - 1):
        jm1, j = j, (2 * s + 1) * inv_y * j - jm1
        res = jnp.where(lcol == s + 1, j, res)

    zc = _sincos(a)[1]
    z2 = zc * zc
    z3 = z2 * zc
    z4 = z2 * z2
    z5 = z3 * z2
    z6 = z3 * z3
    zp = jnp.concatenate(
        [
            jnp.ones_like(zc)[None, :], zc[None, :], z2[None, :], z3[None, :],
            z4[None, :], z5[None, :], z6[None, :], jnp.zeros_like(zc)[None, :],
        ],
        axis=0,
    )
    cb = jax.lax.dot(p_ref[:, :], zp, preferred_element_type=jnp.float32)

    full = res * n_ref[:] * env[None, :] * cb
    o_ref[:, sl] = full[:NCOL, :]


def kernel(dist, angle, idx_kj):
    E = dist.shape[0]
    T = angle.shape[0]

    sc_gather = _make_sc_gather(E, T)
    dist_t = sc_gather(dist, idx_kj)

    BT = 2048
    out = pl.pallas_call(
        _basis_body,
        grid=(pl.cdiv(T, BT),),
        in_specs=[
            pl.BlockSpec((NPAD, 1), lambda i: (0, 0)),
            pl.BlockSpec((NPAD, 1), lambda i: (0, 0)),
            pl.BlockSpec((NPAD, 8), lambda i: (0, 0)),
            pl.BlockSpec((BT,), lambda i: (i,)),
            pl.BlockSpec((BT,), lambda i: (i,)),
        ],
        out_specs=pl.BlockSpec((NCOL, BT), lambda i: (0, i)),
        out_shape=jax.ShapeDtypeStruct((NCOL, T), jnp.float32),
    )(
        jnp.asarray(_ZPAD)[:, None],
        jnp.asarray(_NORMPAD)[:, None],
        jnp.asarray(_PLPAD),
        dist_t,
        angle,
    )
    return out.T
